# baseline (speedup 1.0000x reference)
import functools

import jax
import jax.numpy as jnp
from jax import lax
from jax.experimental import pallas as pl
from jax.experimental.pallas import tpu as pltpu
from jax.experimental.pallas import tpu_sc as plsc

N_CLS = 1000
N_CTX = 16
CTX_DIM = 512
SEQ = 77
SUF = SEQ - 1 - N_CTX
B = 4096

PFX_W = CTX_DIM
CTX_W = N_CTX * CTX_DIM
SFX_W = SUF * CTX_DIM
ROW_W = SEQ * CTX_DIM

NC = 2
NS = 16
NW = NC * NS
BPW = B // NW

_mesh = plsc.VectorSubcoreMesh(core_axis_name="c", subcore_axis_name="s")


@functools.partial(
    pl.kernel,
    out_type=jax.ShapeDtypeStruct((B, ROW_W), jnp.float32),
    mesh=_mesh,
    scratch_types=[
        pltpu.VMEM((BPW,), jnp.int32),
        pltpu.VMEM((BPW * 8,), jnp.int32),
        pltpu.VMEM((1, CTX_W), jnp.float32),
        pltpu.VMEM((BPW, PFX_W), jnp.float32),
        pltpu.VMEM((1, SFX_W), jnp.float32),
        pltpu.SemaphoreType.DMA,
    ],
)
def _prompt_kernel(pfx_hbm, sfx_hbm, ctx_hbm, idx_hbm, idx8_hbm, out_hbm,
                   idx_v, idx8_v, ctx_v, pfx_v, sfx_v, sem):
    wid = lax.axis_index("s") * NC + lax.axis_index("c")
    base = wid * BPW
    pltpu.sync_copy(idx_hbm.at[pl.ds(base, BPW)], idx_v)
    pltpu.sync_copy(idx8_hbm.at[pl.ds(base * 8, BPW * 8)], idx8_v)
    pltpu.sync_copy(ctx_hbm, ctx_v)
    pltpu.async_copy(pfx_hbm.at[idx_v], pfx_v, sem).wait()
    pltpu.sync_copy(pfx_v, out_hbm.at[pl.ds(base, BPW), pl.ds(0, PFX_W)])

    def body(i, carry):
        b = base + i
        pltpu.async_copy(sfx_hbm.at[idx8_v.at[pl.ds(i * 8, 1)]], sfx_v, sem).wait()
        pltpu.sync_copy(ctx_v, out_hbm.at[pl.ds(b, 1), pl.ds(PFX_W, CTX_W)])
        pltpu.sync_copy(sfx_v, out_hbm.at[pl.ds(b, 1), pl.ds(PFX_W + CTX_W, SFX_W)])
        return carry

    lax.fori_loop(0, BPW, body, 0)


def kernel(ctx, token_prefix, token_suffix, compare_idx):
    pfx = token_prefix.reshape(N_CLS, PFX_W)
    sfx = token_suffix.reshape(N_CLS, SFX_W)
    ctx2 = ctx.reshape(1, CTX_W)
    idx = compare_idx.astype(jnp.int32)
    idx8 = jnp.repeat(idx, 8)
    out = _prompt_kernel(pfx, sfx, ctx2, idx, idx8)
    return out.reshape(B, SEQ, CTX_DIM)


# speedup vs baseline: 1.0455x; 1.0455x over previous
import functools

import jax
import jax.numpy as jnp
from jax import lax
from jax.experimental import pallas as pl
from jax.experimental.pallas import tpu as pltpu
from jax.experimental.pallas import tpu_sc as plsc

N_CLS = 1000
N_CTX = 16
CTX_DIM = 512
SEQ = 77
SUF = SEQ - 1 - N_CTX
B = 4096

PFX_W = CTX_DIM
CTX_W = N_CTX * CTX_DIM
SFX_W = SUF * CTX_DIM
ROW_W = SEQ * CTX_DIM

NC = 2
NS = 16
NW = NC * NS
BPW = B // NW

_mesh = plsc.VectorSubcoreMesh(core_axis_name="c", subcore_axis_name="s")


@functools.partial(
    pl.kernel,
    out_type=jax.ShapeDtypeStruct((B, ROW_W), jnp.float32),
    mesh=_mesh,
    scratch_types=[
        pltpu.VMEM((BPW,), jnp.int32),
        pltpu.VMEM((BPW * 8,), jnp.int32),
        pltpu.VMEM((2, CTX_W), jnp.float32),
        pltpu.VMEM((32, PFX_W), jnp.float32),
        pltpu.VMEM((1, SFX_W), jnp.float32),
        pltpu.VMEM((1, SFX_W), jnp.float32),
        pltpu.SemaphoreType.DMA,
        pltpu.SemaphoreType.DMA,
    ],
)
def _prompt_kernel(pfx_hbm, sfx_hbm, ctx_hbm, idx_hbm, idx8_hbm, out_hbm,
                   idx_v, idx8_v, ctx_v, pfx_v, sfx0_v, sfx1_v, sem0, sem1):
    wid = lax.axis_index("s") * NC + lax.axis_index("c")
    base = wid * BPW
    pltpu.sync_copy(idx_hbm.at[pl.ds(base, BPW)], idx_v)
    pltpu.sync_copy(idx8_hbm.at[pl.ds(base * 8, BPW * 8)], idx8_v)
    pltpu.sync_copy(ctx_hbm, ctx_v.at[pl.ds(0, 1)])
    pltpu.sync_copy(ctx_hbm, ctx_v.at[pl.ds(1, 1)])

    for k in range(4):
        pltpu.async_copy(pfx_hbm.at[idx_v.at[pl.ds(k * 32, 32)]], pfx_v, sem0).wait()
        pltpu.sync_copy(
            pfx_v, out_hbm.at[pl.ds(base + k * 32, 32), pl.ds(0, PFX_W)])

    def _gather(i, buf, sem):
        pltpu.async_copy(sfx_hbm.at[idx8_v.at[pl.ds(i * 8, 1)]], buf, sem)

    def _drain(buf, sem):
        pltpu.make_async_copy(sfx_hbm.at[pl.ds(0, 1)], buf, sem).wait()

    _gather(0, sfx0_v, sem0)

    def body(j, carry):
        i0 = 2 * j
        i1 = i0 + 1
        _gather(i1, sfx1_v, sem1)
        _drain(sfx0_v, sem0)
        pltpu.sync_copy(
            sfx0_v, out_hbm.at[pl.ds(base + i0, 1), pl.ds(PFX_W + CTX_W, SFX_W)])
        _gather(lax.rem(i0 + 2, BPW), sfx0_v, sem0)
        pltpu.sync_copy(
            ctx_v, out_hbm.at[pl.ds(base + i0, 2), pl.ds(PFX_W, CTX_W)])
        _drain(sfx1_v, sem1)
        pltpu.sync_copy(
            sfx1_v, out_hbm.at[pl.ds(base + i1, 1), pl.ds(PFX_W + CTX_W, SFX_W)])
        return carry

    lax.fori_loop(0, BPW // 2, body, 0)
    _drain(sfx0_v, sem0)


def kernel(ctx, token_prefix, token_suffix, compare_idx):
    pfx = token_prefix.reshape(N_CLS, PFX_W)
    sfx = token_suffix.reshape(N_CLS, SFX_W)
    ctx2 = ctx.reshape(1, CTX_W)
    idx = compare_idx.astype(jnp.int32)
    idx8 = jnp.repeat(idx, 8)
    out = _prompt_kernel(pfx, sfx, ctx2, idx, idx8)
    return out.reshape(B, SEQ, CTX_DIM)
